# baseline (speedup 1.0000x reference)
import functools

import jax
import jax.numpy as jnp
from jax import lax
from jax.experimental import pallas as pl
from jax.experimental.pallas import tpu as pltpu
from jax.experimental.pallas import tpu_sc as plsc

_ROWS, _COLS = 8192, 4096
_TC_ROWS = 4608
_SC_ROWS = _ROWS - _TC_ROWS
_BLOCK_ROWS = 512

_NW = 32
_ROWS_PER_W = _SC_ROWS // _NW
_CHUNK_ROWS = 8
_NCHUNK = _ROWS_PER_W // _CHUNK_ROWS
_NBUF = 2

_MESH = plsc.VectorSubcoreMesh(core_axis_name="c", subcore_axis_name="s")


def _tc_body(x_ref, o_ref):
    o_ref[...] = x_ref[...]


def _tc_copy(x):
    return pl.pallas_call(
        _tc_body,
        out_shape=jax.ShapeDtypeStruct((_TC_ROWS, _COLS), x.dtype),
        grid=(_TC_ROWS // _BLOCK_ROWS,),
        in_specs=[pl.BlockSpec((_BLOCK_ROWS, _COLS), lambda i: (i, 0))],
        out_specs=pl.BlockSpec((_BLOCK_ROWS, _COLS), lambda i: (i, 0)),
    )(x)


@functools.partial(
    pl.kernel,
    out_type=jax.ShapeDtypeStruct((_SC_ROWS, _COLS), jnp.float32),
    mesh=_MESH,
    scratch_types=[
        pltpu.VMEM((_NBUF, _CHUNK_ROWS, _COLS), jnp.float32),
        pltpu.SemaphoreType.DMA((_NBUF,)),
        pltpu.SemaphoreType.DMA((_NBUF,)),
    ],
)
def _sc_copy(x_hbm, o_hbm, buf, in_sems, out_sems):
    wid = lax.axis_index("s") * 2 + lax.axis_index("c")
    base = wid * _ROWS_PER_W

    def in_slice(i):
        return pl.ds(_TC_ROWS + base + i * _CHUNK_ROWS, _CHUNK_ROWS)

    def out_slice(i):
        return pl.ds(base + i * _CHUNK_ROWS, _CHUNK_ROWS)

    for b in range(_NBUF):
        pltpu.async_copy(x_hbm.at[in_slice(b)], buf.at[b], in_sems.at[b])

    for i in range(_NCHUNK):
        b = i % _NBUF
        pltpu.make_async_copy(x_hbm.at[in_slice(i)], buf.at[b],
                              in_sems.at[b]).wait()
        pltpu.async_copy(buf.at[b], o_hbm.at[out_slice(i)], out_sems.at[b])
        if i + _NBUF < _NCHUNK:
            pltpu.make_async_copy(buf.at[b], o_hbm.at[out_slice(i)],
                                  out_sems.at[b]).wait()
            pltpu.async_copy(x_hbm.at[in_slice(i + _NBUF)], buf.at[b],
                             in_sems.at[b])

    for i in range(_NCHUNK - _NBUF, _NCHUNK):
        b = i % _NBUF
        pltpu.make_async_copy(buf.at[b], o_hbm.at[out_slice(i)],
                              out_sems.at[b]).wait()


def kernel(x):
    top = _tc_copy(x)
    bottom = _sc_copy(x)
    return jnp.concatenate([top, bottom], axis=0)


# speedup vs baseline: 2.2444x; 2.2444x over previous
import jax
import jax.numpy as jnp
from jax.experimental import pallas as pl
from jax.experimental.pallas import tpu as pltpu

_BLOCK_ROWS = 512


def _copy_kernel(x_ref, o_ref):
    o_ref[...] = x_ref[...]


def kernel(x):
    rows, cols = x.shape
    grid = (rows // _BLOCK_ROWS,)
    return pl.pallas_call(
        _copy_kernel,
        out_shape=jax.ShapeDtypeStruct(x.shape, x.dtype),
        grid=grid,
        in_specs=[pl.BlockSpec((_BLOCK_ROWS, cols), lambda i: (i, 0))],
        out_specs=pl.BlockSpec((_BLOCK_ROWS, cols), lambda i: (i, 0)),
    )(x)
